# Initial kernel scaffold; baseline (speedup 1.0000x reference)
import jax
import jax.numpy as jnp
from jax.experimental import pallas as pl


def kernel(x, W1, b1, W2, b2, edge_index):
    raise NotImplementedError("write your pallas kernel here")


# speedup vs baseline: 216.2769x; 216.2769x over previous
import functools

import jax
import jax.numpy as jnp
from jax import lax
from jax.experimental import pallas as pl
from jax.experimental.pallas import tpu as pltpu
from jax.experimental.pallas import tpu_sc as plsc

NC, NS, L = 2, 16, 16
NW = NC * NS
SUBLEN = 1568


def _rsqrt_nr(d):
    i = plsc.bitcast(d, jnp.int32)
    i = jnp.int32(0x5F3759DF) - (i >> 1)
    y = plsc.bitcast(i, jnp.float32)
    for _ in range(3):
        y = y * (1.5 - 0.5 * d * y * y)
    return y


def _splat(param_v, idx):
    return plsc.load_gather(param_v, [jnp.full((L,), idx, jnp.int32)])


def kernel(x, W1, b1, W2, b2, edge_index):
    N = x.shape[1]
    E = edge_index.shape[1]
    f32 = jnp.float32

    NP = -(-(N + 1) // (NW * L)) * (NW * L)
    CHS = NP // NS
    CHG = NP // NW
    RW = -(-E // (128 * NW * 16)) * 16
    RP = RW * NW
    EP = RP * 128
    PAD = EP - E
    FW = RW // 16

    xp = jnp.zeros((NP,), f32).at[:N].set(x[0, :, 0].astype(f32))
    src = edge_index[0]
    dst = edge_index[1]
    if PAD:
        src = jnp.concatenate([src, jnp.full((PAD,), N, jnp.int32)])
        dst = jnp.concatenate([dst, jnp.arange(PAD, dtype=jnp.int32) % N])
    srcR = src.reshape(RP, 128)
    dstR = dst.reshape(RP, 128)
    ones_t = (jnp.arange(NP) < N).astype(f32)
    zeros_v = jnp.zeros((NP,), f32)
    params = (jnp.zeros((32,), f32)
              .at[0:8].set(W1.reshape(8).astype(f32))
              .at[8:16].set(b1.astype(f32))
              .at[16:24].set(W2.reshape(8).astype(f32))
              .at[24].set(b2.reshape(())[()].astype(f32)))

    mesh = plsc.VectorSubcoreMesh(core_axis_name="c", subcore_axis_name="s")
    cparams = pltpu.CompilerParams(needs_layout_passes=False)

    def edge_pass(srcR_h, dstR_h, table_v, acc_sp, src_b, dst_b, msg_b, sem, wid):
        row0 = wid * RW

        def window(base, nrows):
            pltpu.sync_copy(srcR_h.at[pl.ds(base, nrows)], src_b.at[pl.ds(0, nrows)])
            pltpu.sync_copy(dstR_h.at[pl.ds(base, nrows)], dst_b.at[pl.ds(0, nrows)])
            descs = []
            for j in range(nrows):
                for i in range(8):
                    sl = pl.ds(i * 16, 16)
                    msg_b[j, sl] = plsc.load_gather(table_v, [src_b[j, sl]])
                descs.append(
                    pltpu.async_copy(msg_b.at[j], acc_sp.at[dst_b.at[j]], sem,
                                     add=True))
            for d in descs:
                d.wait()

        @pl.loop(0, FW)
        def _(w):
            window(row0 + w * 16, 16)

    def acc_out(acc_sp, out_h, cid, sid):
        sl = pl.ds(sid * CHS, CHS)
        pltpu.sync_copy(acc_sp.at[sl], out_h.at[pl.ds(cid * NP + sid * CHS, CHS)])

    scatter_scratch = [
        pltpu.VMEM((NP,), f32),
        pltpu.VMEM((16, 128), jnp.int32),
        pltpu.VMEM((16, 128), jnp.int32),
        pltpu.VMEM((16, 128), f32),
        pltpu.VMEM_SHARED((NP,), f32),
        pltpu.SemaphoreType.DMA,
    ]

    @functools.partial(
        pl.kernel,
        out_type=jax.ShapeDtypeStruct((NC * NP,), f32),
        mesh=mesh,
        compiler_params=cparams,
        scratch_types=scatter_scratch,
    )
    def k_deg(ones_h, srcR_h, dstR_h, zeros_h, degp_h,
              table_v, src_b, dst_b, msg_b, acc_sp, sem):
        cid = lax.axis_index("c")
        sid = lax.axis_index("s")
        wid = cid * NS + sid

        @pl.when(sid == 0)
        def _():
            pltpu.sync_copy(zeros_h, acc_sp)

        pltpu.sync_copy(ones_h, table_v)
        plsc.subcore_barrier()
        edge_pass(srcR_h, dstR_h, table_v, acc_sp, src_b, dst_b, msg_b, sem, wid)
        plsc.subcore_barrier()
        acc_out(acc_sp, degp_h, cid, sid)

    @functools.partial(
        pl.kernel,
        out_type=(jax.ShapeDtypeStruct((NC * NP,), f32),
                  jax.ShapeDtypeStruct((NC * NP,), f32)),
        mesh=mesh,
        compiler_params=cparams,
        scratch_types=scatter_scratch + [
            pltpu.VMEM((SUBLEN,), f32),
            pltpu.VMEM((SUBLEN,), f32),
            pltpu.VMEM((SUBLEN,), f32),
            pltpu.VMEM((SUBLEN,), f32),
        ],
    )
    def k_mv1(degp_h, xp_h, srcR_h, dstR_h, zeros_h, acc1_h, wt_h,
              table_v, src_b, dst_b, msg_b, acc_sp, sem,
              bd0, bd1, bx, bw):
        cid = lax.axis_index("c")
        sid = lax.axis_index("s")
        wid = cid * NS + sid

        @pl.when(sid == 0)
        def _():
            pltpu.sync_copy(zeros_h, acc_sp)

        for sub in range(CHS // SUBLEN):
            base = sid * CHS + sub * SUBLEN
            bsl = pl.ds(base, SUBLEN)
            pltpu.sync_copy(degp_h.at[pl.ds(base, SUBLEN)], bd0)
            pltpu.sync_copy(degp_h.at[pl.ds(NP + base, SUBLEN)], bd1)
            pltpu.sync_copy(xp_h.at[bsl], bx)

            @pl.loop(0, SUBLEN // L)
            def _(k):
                sl = pl.ds(k * L, L)
                dv = _rsqrt_nr(bd0[sl] + bd1[sl] + 1.0)
                bw[sl] = dv * bx[sl]

            pltpu.sync_copy(bw, wt_h.at[pl.ds(cid * NP + base, SUBLEN)])

        plsc.subcore_barrier()
        pltpu.sync_copy(wt_h.at[pl.ds(cid * NP, NP)], table_v)
        edge_pass(srcR_h, dstR_h, table_v, acc_sp, src_b, dst_b, msg_b, sem, wid)
        plsc.subcore_barrier()
        acc_out(acc_sp, acc1_h, cid, sid)

    @functools.partial(
        pl.kernel,
        out_type=(jax.ShapeDtypeStruct((NC * NP,), f32),
                  jax.ShapeDtypeStruct((NC * NP,), f32)),
        mesh=mesh,
        compiler_params=cparams,
        scratch_types=scatter_scratch + [
            pltpu.VMEM((SUBLEN,), f32),
            pltpu.VMEM((SUBLEN,), f32),
            pltpu.VMEM((SUBLEN,), f32),
            pltpu.VMEM((SUBLEN,), f32),
            pltpu.VMEM((SUBLEN,), f32),
            pltpu.VMEM((SUBLEN,), f32),
            pltpu.VMEM((32,), f32),
        ],
    )
    def k_mv2(degp_h, acc1_h, xp_h, params_h, srcR_h, dstR_h, zeros_h,
              acc2_h, w2t_h,
              table_v, src_b, dst_b, msg_b, acc_sp, sem,
              bd0, bd1, ba0, ba1, bx, bw, param_v):
        cid = lax.axis_index("c")
        sid = lax.axis_index("s")
        wid = cid * NS + sid

        @pl.when(sid == 0)
        def _():
            pltpu.sync_copy(zeros_h, acc_sp)

        pltpu.sync_copy(params_h, param_v)
        w1v = [_splat(param_v, c) for c in range(8)]
        b1v = [_splat(param_v, 8 + c) for c in range(8)]
        w2v = [_splat(param_v, 16 + c) for c in range(8)]

        for sub in range(CHS // SUBLEN):
            base = sid * CHS + sub * SUBLEN
            bsl = pl.ds(base, SUBLEN)
            pltpu.sync_copy(degp_h.at[pl.ds(base, SUBLEN)], bd0)
            pltpu.sync_copy(degp_h.at[pl.ds(NP + base, SUBLEN)], bd1)
            pltpu.sync_copy(acc1_h.at[pl.ds(base, SUBLEN)], ba0)
            pltpu.sync_copy(acc1_h.at[pl.ds(NP + base, SUBLEN)], ba1)
            pltpu.sync_copy(xp_h.at[bsl], bx)

            @pl.loop(0, SUBLEN // L)
            def _(k):
                sl = pl.ds(k * L, L)
                dv = _rsqrt_nr(bd0[sl] + bd1[sl] + 1.0)
                w = dv * bx[sl]
                s = dv * (ba0[sl] + ba1[sl] + w)
                t = jnp.zeros((L,), f32)
                for c in range(8):
                    t = t + jnp.maximum(s * w1v[c] + b1v[c], 0.0) * w2v[c]
                w2 = dv * t
                gi = base + k * L + lax.iota(jnp.int32, L)
                bw[sl] = jnp.where(gi < N, w2, 0.0)

            pltpu.sync_copy(bw, w2t_h.at[pl.ds(cid * NP + base, SUBLEN)])

        plsc.subcore_barrier()
        pltpu.sync_copy(w2t_h.at[pl.ds(cid * NP, NP)], table_v)
        edge_pass(srcR_h, dstR_h, table_v, acc_sp, src_b, dst_b, msg_b, sem, wid)
        plsc.subcore_barrier()
        acc_out(acc_sp, acc2_h, cid, sid)

    @functools.partial(
        pl.kernel,
        out_type=jax.ShapeDtypeStruct((NP,), f32),
        mesh=mesh,
        compiler_params=cparams,
        scratch_types=[
            pltpu.VMEM((SUBLEN,), f32),
            pltpu.VMEM((SUBLEN,), f32),
            pltpu.VMEM((SUBLEN,), f32),
            pltpu.VMEM((SUBLEN,), f32),
            pltpu.VMEM((SUBLEN,), f32),
            pltpu.VMEM((SUBLEN,), f32),
            pltpu.VMEM((32,), f32),
        ],
    )
    def k_fin(degp_h, acc2_h, w2t_h, params_h, out_h,
              bd0, bd1, ba0, ba1, bw, bo, param_v):
        cid = lax.axis_index("c")
        sid = lax.axis_index("s")
        wid = cid * NS + sid

        pltpu.sync_copy(params_h, param_v)
        b2v = _splat(param_v, 24)

        for sub in range(CHG // SUBLEN):
            base = wid * CHG + sub * SUBLEN
            bsl = pl.ds(base, SUBLEN)
            pltpu.sync_copy(degp_h.at[pl.ds(base, SUBLEN)], bd0)
            pltpu.sync_copy(degp_h.at[pl.ds(NP + base, SUBLEN)], bd1)
            pltpu.sync_copy(acc2_h.at[pl.ds(base, SUBLEN)], ba0)
            pltpu.sync_copy(acc2_h.at[pl.ds(NP + base, SUBLEN)], ba1)
            pltpu.sync_copy(w2t_h.at[bsl], bw)

            @pl.loop(0, SUBLEN // L)
            def _(k):
                sl = pl.ds(k * L, L)
                dv = _rsqrt_nr(bd0[sl] + bd1[sl] + 1.0)
                u = dv * (ba0[sl] + ba1[sl] + bw[sl]) + b2v
                bo[sl] = 1.0 / (1.0 + jnp.exp(-u))

            pltpu.sync_copy(bo, out_h.at[bsl])

    degp = k_deg(ones_t, srcR, dstR, zeros_v)
    acc1, wt = k_mv1(degp, xp, srcR, dstR, zeros_v)
    acc2, w2t = k_mv2(degp, acc1, xp, params, srcR, dstR, zeros_v)
    out_pad = k_fin(degp, acc2, w2t, params)
    return out_pad[:N].reshape(1, N, 1).astype(x.dtype)
